# baseline (speedup 1.0000x reference)
import jax
import jax.numpy as jnp
from jax.experimental import pallas as pl

VOCAB = 1000


def _onehot_block(ids_ref, out_ref):
    ids = ids_ref[0, 0, :]
    iota = jax.lax.broadcasted_iota(jnp.int32, (VOCAB, ids.shape[0]), 0)
    out_ref[0, :, :] = jnp.where(iota == ids[None, :], 1.0, 0.0).astype(jnp.float32)


def kernel(input_ids) -> jnp.ndarray:
    B, L = input_ids.shape
    ids_t = input_ids.T.reshape(L, 1, B).astype(jnp.int32)
    out = pl.pallas_call(
        _onehot_block,
        grid=(L,),
        in_specs=[pl.BlockSpec((1, 1, B), lambda i: (i, 0, 0))],
        out_specs=pl.BlockSpec((1, VOCAB, B), lambda i: (i, 0, 0)),
        out_shape=jax.ShapeDtypeStruct((L, VOCAB, B), jnp.float32),
    )(ids_t)
    return out.transpose(2, 0, 1)


# speedup vs baseline: 1.0258x; 1.0258x over previous
import jax
import jax.numpy as jnp
from jax.experimental import pallas as pl

VOCAB = 1000


def _onehot_block(ids_ref, out_ref):
    ids = ids_ref[pl.program_id(0), :]
    iota = jax.lax.broadcasted_iota(jnp.int32, (VOCAB, ids.shape[0]), 0)
    out_ref[0, :, :] = jnp.where(iota == ids[None, :], 1.0, 0.0)


def kernel(input_ids) -> jnp.ndarray:
    B, L = input_ids.shape
    ids_t = input_ids.T.astype(jnp.int32)
    out = pl.pallas_call(
        _onehot_block,
        grid=(L,),
        in_specs=[pl.BlockSpec((L, B), lambda i: (0, 0))],
        out_specs=pl.BlockSpec((1, VOCAB, B), lambda i: (i, 0, 0)),
        out_shape=jax.ShapeDtypeStruct((L, VOCAB, B), jnp.float32),
    )(ids_t)
    return out.transpose(2, 0, 1)
